# baseline (speedup 1.0000x reference)
import jax
import jax.numpy as jnp
from jax.experimental import pallas as pl
from jax.experimental.pallas import tpu as pltpu

_BB = 4
_DEPTH = 8


def _sampler_body(x_hbm, probs_hbm, logprobs_hbm, tok_ref, slp_ref,
                  xs, ps, lps, in_sems, op_sems, ol_sems):
    batch = x_hbm.shape[0]
    vocab = x_hbm.shape[1]
    nchunk = batch // _BB

    def in_copy(c, slot):
        return pltpu.make_async_copy(
            x_hbm.at[pl.ds(c * _BB, _BB), :], xs.at[slot], in_sems.at[slot])

    out_copies = {}

    for d in range(min(_DEPTH, nchunk)):
        in_copy(d, d).start()

    for c in range(nchunk):
        slot = c % _DEPTH
        in_copy(c, slot).wait()
        if c >= _DEPTH:
            for cp in out_copies.pop(c - _DEPTH):
                cp.wait()
        x = xs[slot]
        m = jnp.max(x, axis=-1, keepdims=True)
        xm = x - m
        e = jnp.exp(xm)
        s = jnp.sum(e, axis=-1, keepdims=True)
        ps[slot] = e * (1.0 / s)
        ls = jnp.log(s)
        lps[slot] = xm - ls
        idx = jax.lax.broadcasted_iota(jnp.int32, x.shape, 1)
        cand = jnp.where(x == m, idx, vocab)
        tok_ref[pl.ds(c * _BB, _BB), :] = jnp.min(cand, axis=-1, keepdims=True)
        slp_ref[pl.ds(c * _BB, _BB), :] = -ls
        cp_p = pltpu.make_async_copy(
            ps.at[slot], probs_hbm.at[pl.ds(c * _BB, _BB), :], op_sems.at[slot])
        cp_l = pltpu.make_async_copy(
            lps.at[slot], logprobs_hbm.at[pl.ds(c * _BB, _BB), :], ol_sems.at[slot])
        cp_p.start()
        cp_l.start()
        out_copies[c] = (cp_p, cp_l)
        if c + _DEPTH < nchunk:
            in_copy(c + _DEPTH, slot).start()

    for cps in out_copies.values():
        for cp in cps:
            cp.wait()


def kernel(logits):
    batch, vocab = logits.shape
    out = pl.pallas_call(
        _sampler_body,
        in_specs=[pl.BlockSpec(memory_space=pl.ANY)],
        out_specs=[
            pl.BlockSpec(memory_space=pl.ANY),
            pl.BlockSpec(memory_space=pl.ANY),
            pl.BlockSpec(memory_space=pltpu.MemorySpace.VMEM),
            pl.BlockSpec(memory_space=pltpu.MemorySpace.VMEM),
        ],
        out_shape=[
            jax.ShapeDtypeStruct((batch, vocab), jnp.float32),
            jax.ShapeDtypeStruct((batch, vocab), jnp.float32),
            jax.ShapeDtypeStruct((batch, 1), jnp.int32),
            jax.ShapeDtypeStruct((batch, 1), jnp.float32),
        ],
        scratch_shapes=[
            pltpu.VMEM((_DEPTH, _BB, vocab), jnp.float32),
            pltpu.VMEM((_DEPTH, _BB, vocab), jnp.float32),
            pltpu.VMEM((_DEPTH, _BB, vocab), jnp.float32),
            pltpu.SemaphoreType.DMA((_DEPTH,)),
            pltpu.SemaphoreType.DMA((_DEPTH,)),
            pltpu.SemaphoreType.DMA((_DEPTH,)),
        ],
    )(logits.astype(jnp.float32))
    probs, logprobs, next_tokens, sample_logprobs = out
    return probs, logprobs, next_tokens.reshape(batch), sample_logprobs


# speedup vs baseline: 1.0752x; 1.0752x over previous
import jax
import jax.numpy as jnp
from jax.experimental import pallas as pl
from jax.experimental.pallas import tpu as pltpu

_BB = 8


def _block_body(x_ref, probs_ref, logprobs_ref, tok_ref, slp_ref):
    x = x_ref[...]
    vocab = x.shape[-1]
    m = jnp.max(x, axis=-1, keepdims=True)
    xm = x - m
    e = jnp.exp(xm)
    s = jnp.sum(e, axis=-1, keepdims=True)
    probs_ref[...] = e * (1.0 / s)
    ls = jnp.log(s)
    logprobs_ref[...] = xm - ls
    idx = jax.lax.broadcasted_iota(jnp.int32, x.shape, 1)
    cand = jnp.where(x == m, idx, vocab)
    tok_ref[...] = jnp.min(cand, axis=-1, keepdims=True)
    slp_ref[...] = -ls


def kernel(logits):
    batch, vocab = logits.shape
    bb = _BB
    mesh = pltpu.create_tensorcore_mesh("core", num_cores=2)

    @pl.kernel(
        out_type=[
            jax.ShapeDtypeStruct((batch, vocab), jnp.float32),
            jax.ShapeDtypeStruct((batch, vocab), jnp.float32),
            jax.ShapeDtypeStruct((batch, 1), jnp.int32),
            jax.ShapeDtypeStruct((batch, 1), jnp.float32),
        ],
        mesh=mesh,
    )
    def run(x_hbm, p_hbm, l_hbm, t_hbm, s_hbm):
        pltpu.emit_pipeline(
            _block_body,
            grid=(batch // bb,),
            in_specs=[pl.BlockSpec((bb, vocab), lambda i: (i, 0))],
            out_specs=[
                pl.BlockSpec((bb, vocab), lambda i: (i, 0)),
                pl.BlockSpec((bb, vocab), lambda i: (i, 0)),
                pl.BlockSpec((bb, 1), lambda i: (i, 0)),
                pl.BlockSpec((bb, 1), lambda i: (i, 0)),
            ],
            core_axis_name="core",
            dimension_semantics=(pltpu.PARALLEL,),
        )(x_hbm, p_hbm, l_hbm, t_hbm, s_hbm)

    probs, logprobs, next_tokens, sample_logprobs = run(logits.astype(jnp.float32))
    return probs, logprobs, next_tokens.reshape(batch), sample_logprobs


# speedup vs baseline: 1.1246x; 1.0459x over previous
import jax
import jax.numpy as jnp
from jax.experimental import pallas as pl


def _sampler_body(x_ref, probs_ref, logprobs_ref, tok_ref, slp_ref):
    x = x_ref[...]
    vocab = x.shape[-1]
    m = jnp.max(x, axis=-1, keepdims=True)
    xm = x - m
    e = jnp.exp(xm)
    s = jnp.sum(e, axis=-1, keepdims=True)
    probs_ref[...] = e * (1.0 / s)
    ls = jnp.log(s)
    logprobs_ref[...] = xm - ls
    idx = jax.lax.broadcasted_iota(jnp.int32, x.shape, 1)
    cand = jnp.where(x == m, idx, vocab)
    tok_ref[...] = jnp.min(cand, axis=-1, keepdims=True)
    slp_ref[...] = -ls


def kernel(logits):
    batch, vocab = logits.shape
    bb = 16
    grid = (batch // bb,)
    out = pl.pallas_call(
        _sampler_body,
        grid=grid,
        in_specs=[pl.BlockSpec((bb, vocab), lambda i: (i, 0))],
        out_specs=[
            pl.BlockSpec((bb, vocab), lambda i: (i, 0)),
            pl.BlockSpec((bb, vocab), lambda i: (i, 0)),
            pl.BlockSpec((bb, 1), lambda i: (i, 0)),
            pl.BlockSpec((bb, 1), lambda i: (i, 0)),
        ],
        out_shape=[
            jax.ShapeDtypeStruct((batch, vocab), jnp.float32),
            jax.ShapeDtypeStruct((batch, vocab), jnp.float32),
            jax.ShapeDtypeStruct((batch, 1), jnp.int32),
            jax.ShapeDtypeStruct((batch, 1), jnp.float32),
        ],
    )(logits.astype(jnp.float32))
    probs, logprobs, next_tokens, sample_logprobs = out
    return probs, logprobs, next_tokens.reshape(batch), sample_logprobs


# speedup vs baseline: 1.1359x; 1.0100x over previous
import jax
import jax.numpy as jnp
from jax.experimental import pallas as pl
from jax.experimental.pallas import tpu as pltpu

_BB = 16
_DEPTH = 2


def _sampler_body(x_hbm, probs_hbm, logprobs_hbm, tok_ref, slp_ref,
                  xs, ps, lps, in_sems, op_sems, ol_sems):
    batch = x_hbm.shape[0]
    vocab = x_hbm.shape[1]
    nchunk = batch // _BB

    def in_copy(c, slot):
        return pltpu.make_async_copy(
            x_hbm.at[pl.ds(c * _BB, _BB), :], xs.at[slot], in_sems.at[slot])

    out_copies = {}

    for d in range(min(_DEPTH, nchunk)):
        in_copy(d, d).start()

    for c in range(nchunk):
        slot = c % _DEPTH
        in_copy(c, slot).wait()
        if c >= _DEPTH:
            for cp in out_copies.pop(c - _DEPTH):
                cp.wait()
        x = xs[slot]
        m = jnp.max(x, axis=-1, keepdims=True)
        xm = x - m
        e = jnp.exp(xm)
        s = jnp.sum(e, axis=-1, keepdims=True)
        ps[slot] = e * (1.0 / s)
        cp_p = pltpu.make_async_copy(
            ps.at[slot], probs_hbm.at[pl.ds(c * _BB, _BB), :], op_sems.at[slot])
        cp_p.start()
        ls = jnp.log(s)
        lps[slot] = xm - ls
        cp_l = pltpu.make_async_copy(
            lps.at[slot], logprobs_hbm.at[pl.ds(c * _BB, _BB), :], ol_sems.at[slot])
        cp_l.start()
        idx = jax.lax.broadcasted_iota(jnp.int32, x.shape, 1)
        cand = jnp.where(x == m, idx, vocab)
        tok_ref[pl.ds(c * _BB, _BB), :] = jnp.min(cand, axis=-1, keepdims=True)
        slp_ref[pl.ds(c * _BB, _BB), :] = -ls
        out_copies[c] = (cp_p, cp_l)
        if c + _DEPTH < nchunk:
            in_copy(c + _DEPTH, slot).start()

    for cps in out_copies.values():
        for cp in cps:
            cp.wait()


def kernel(logits):
    batch, vocab = logits.shape
    out = pl.pallas_call(
        _sampler_body,
        in_specs=[pl.BlockSpec(memory_space=pl.ANY)],
        out_specs=[
            pl.BlockSpec(memory_space=pl.ANY),
            pl.BlockSpec(memory_space=pl.ANY),
            pl.BlockSpec(memory_space=pltpu.MemorySpace.VMEM),
            pl.BlockSpec(memory_space=pltpu.MemorySpace.VMEM),
        ],
        out_shape=[
            jax.ShapeDtypeStruct((batch, vocab), jnp.float32),
            jax.ShapeDtypeStruct((batch, vocab), jnp.float32),
            jax.ShapeDtypeStruct((batch, 1), jnp.int32),
            jax.ShapeDtypeStruct((batch, 1), jnp.float32),
        ],
        scratch_shapes=[
            pltpu.VMEM((_DEPTH, _BB, vocab), jnp.float32),
            pltpu.VMEM((_DEPTH, _BB, vocab), jnp.float32),
            pltpu.VMEM((_DEPTH, _BB, vocab), jnp.float32),
            pltpu.SemaphoreType.DMA((_DEPTH,)),
            pltpu.SemaphoreType.DMA((_DEPTH,)),
            pltpu.SemaphoreType.DMA((_DEPTH,)),
        ],
    )(logits.astype(jnp.float32))
    probs, logprobs, next_tokens, sample_logprobs = out
    return probs, logprobs, next_tokens.reshape(batch), sample_logprobs
